# Initial kernel scaffold; baseline (speedup 1.0000x reference)
import jax
import jax.numpy as jnp
from jax.experimental import pallas as pl


def kernel(numerical_input, categorical_inputs, emb_tables, W_bot_0, b_bot_0, W_bot_1, b_bot_1, W_bot_2, b_bot_2, W_top_0, b_top_0, W_top_1, b_top_1, W_top_2, b_top_2, W_top_3, b_top_3, W_top_4, b_top_4):
    raise NotImplementedError("write your pallas kernel here")


# speedup vs baseline: 1.9081x; 1.9081x over previous
import functools

import numpy as np
import jax
import jax.numpy as jnp
from jax import lax
from jax.experimental import pallas as pl
from jax.experimental.pallas import tpu as pltpu
from jax.experimental.pallas import tpu_sc as plsc

_B = 16384
_NF = 26
_V = 100000
_D = 32
_NI = _NF + 1
_BNF = _B * _NF
_NW = 32
_RPW = _BNF // _NW
_CH = 128
_NCH = _RPW // _CH
_R = 512

_TRI = np.array([i * _NI + j for i in range(_NI) for j in range(i)], dtype=np.int32)


def _sc_gather(tables_flat, idx3):
    mesh = plsc.VectorSubcoreMesh(core_axis_name="c", subcore_axis_name="s")

    @functools.partial(
        pl.kernel,
        mesh=mesh,
        out_type=jax.ShapeDtypeStruct((_BNF, _D), jnp.float32),
        compiler_params=pltpu.CompilerParams(use_tc_tiling_on_sc=False),
        scratch_types=[
            pltpu.VMEM((_NCH, _CH), jnp.int32),
            pltpu.VMEM((_CH, _D), jnp.float32),
            pltpu.VMEM((_CH, _D), jnp.float32),
            pltpu.SemaphoreType.DMA,
            pltpu.SemaphoreType.DMA,
        ],
    )
    def k(tab_hbm, idx_hbm, out_hbm, idx_v, buf0, buf1, sem0, sem1):
        wid = lax.axis_index("s") * 2 + lax.axis_index("c")
        base = wid * _RPW
        pltpu.sync_copy(idx_hbm.at[wid], idx_v)
        pltpu.async_copy(tab_hbm.at[idx_v.at[0]], buf0, sem0)
        pltpu.async_copy(tab_hbm.at[idx_v.at[1]], buf1, sem1)

        def body(kk, _):
            c0 = 2 * kk

            def step(buf, sem, c):
                pltpu.make_async_copy(tab_hbm.at[idx_v.at[c]], buf, sem).wait()
                pltpu.sync_copy(buf, out_hbm.at[pl.ds(base + c * _CH, _CH)])

                @pl.when(c + 2 < _NCH)
                def _():
                    pltpu.async_copy(tab_hbm.at[idx_v.at[c + 2]], buf, sem)

            step(buf0, sem0, c0)
            step(buf1, sem1, c0 + 1)
            return ()

        lax.fori_loop(0, _NCH // 2, body, (), unroll=False)

    return k(tables_flat, idx3)


def _tc_body(num_ref, gat_ref, wb0, bb0, wb1, bb1, wb2, bb2,
             w0a, w0i, bt0, wt1, bt1, wt2, bt2, wt3, bt3, wt4, bt4,
             out_ref):
    f32 = jnp.float32
    x = num_ref[...]
    x = jnp.maximum(jnp.dot(x, wb0[...], preferred_element_type=f32) + bb0[...], 0.0)
    x = jnp.maximum(jnp.dot(x, wb1[...], preferred_element_type=f32) + bb1[...], 0.0)
    bot = jnp.maximum(jnp.dot(x, wb2[...], preferred_element_type=f32) + bb2[...], 0.0)
    C = jnp.concatenate([bot, gat_ref[...]], axis=1)
    C3 = C.reshape(_R, _NI, _D)
    inter = lax.dot_general(C3, C3, (((2,), (2,)), ((0,), (0,))),
                            preferred_element_type=f32)
    interf = inter.reshape(_R, _NI * _NI)
    y = jnp.dot(bot, w0a[...], preferred_element_type=f32)
    y = y + jnp.dot(interf, w0i[...], preferred_element_type=f32)
    y = jnp.maximum(y + bt0[...], 0.0)
    y = jnp.maximum(jnp.dot(y, wt1[...], preferred_element_type=f32) + bt1[...], 0.0)
    y = jnp.maximum(jnp.dot(y, wt2[...], preferred_element_type=f32) + bt2[...], 0.0)
    y = jnp.maximum(jnp.dot(y, wt3[...], preferred_element_type=f32) + bt3[...], 0.0)
    out_ref[...] = jnp.dot(y, wt4[...], preferred_element_type=f32) + bt4[...]


def kernel(numerical_input, categorical_inputs, emb_tables,
           W_bot_0, b_bot_0, W_bot_1, b_bot_1, W_bot_2, b_bot_2,
           W_top_0, b_top_0, W_top_1, b_top_1, W_top_2, b_top_2,
           W_top_3, b_top_3, W_top_4, b_top_4):
    offs = (jnp.arange(_NF, dtype=jnp.int32) * _V)[None, :]
    idx3 = (categorical_inputs + offs).reshape(_NW, _NCH, _CH)
    gathered = _sc_gather(emb_tables.reshape(_NF * _V, _D), idx3)
    gat2 = gathered.reshape(_B, _NF * _D)

    w0a = W_top_0[:_D]
    w0i = jnp.zeros((_NI * _NI, W_top_0.shape[1]), jnp.float32)
    w0i = w0i.at[_TRI].set(W_top_0[_D:_D + _TRI.shape[0]])

    row = lambda b: b.reshape(1, -1)
    grid = _B // _R
    full = lambda a: pl.BlockSpec(a.shape, lambda i: (0,) * a.ndim)
    out = pl.pallas_call(
        _tc_body,
        grid=(grid,),
        in_specs=[
            pl.BlockSpec((_R, numerical_input.shape[1]), lambda i: (i, 0)),
            pl.BlockSpec((_R, _NF * _D), lambda i: (i, 0)),
            full(W_bot_0), full(row(b_bot_0)), full(W_bot_1), full(row(b_bot_1)),
            full(W_bot_2), full(row(b_bot_2)),
            full(w0a), full(w0i), full(row(b_top_0)),
            full(W_top_1), full(row(b_top_1)), full(W_top_2), full(row(b_top_2)),
            full(W_top_3), full(row(b_top_3)), full(W_top_4), full(row(b_top_4)),
        ],
        out_specs=pl.BlockSpec((_R, 1), lambda i: (i, 0)),
        out_shape=jax.ShapeDtypeStruct((_B, 1), jnp.float32),
    )(numerical_input, gat2,
      W_bot_0, row(b_bot_0), W_bot_1, row(b_bot_1), W_bot_2, row(b_bot_2),
      w0a, w0i, row(b_top_0), W_top_1, row(b_top_1), W_top_2, row(b_top_2),
      W_top_3, row(b_top_3), W_top_4, row(b_top_4))
    return out
